# Initial kernel scaffold; baseline (speedup 1.0000x reference)
import jax
import jax.numpy as jnp
from jax.experimental import pallas as pl


def kernel(x, demographic, patient_edge_idx, group_edge_idx, pW1, pb1, pW2, pb2, pW3, pb3, plinW, plinb, gW1, gb1, gW2, gb2, gW3, gb3, gW4, gb4):
    raise NotImplementedError("write your pallas kernel here")


# speedup vs baseline: 9.7684x; 9.7684x over previous
import functools

import jax
import jax.numpy as jnp
from jax import lax
from jax.experimental import pallas as pl

B = 128
N = 512
F = 64
PE = 500
GE = 5000
HID = 128
GED = 128
CLIN = 6
NCLS = 2

BP = 8


def _patient_body(x_ref, pe_ref, w1_ref, b1_ref, w2_ref, b2_ref,
                  w3_ref, b3_ref, plw_ref, plb_ref, out_ref):
    iota_n = lax.broadcasted_iota(jnp.int32, (N, PE), 0)
    w1 = w1_ref[...]
    w2 = w2_ref[...]
    w3 = w3_ref[...]
    b1 = b1_ref[...]
    b2 = b2_ref[...]
    b3 = b3_ref[...]
    for p in range(BP):
        s_row = pe_ref[p, 0:1, :]
        d_row = pe_ref[p, 1:2, :]
        od = (d_row == iota_n).astype(jnp.float32)
        os_ = (s_row == iota_n).astype(jnp.float32)
        abar = lax.dot_general(od, os_, (((1,), (1,)), ((), ())),
                               preferred_element_type=jnp.float32)
        deg = jnp.sum(abar, axis=1, keepdims=True) + 1.0
        dis = lax.rsqrt(deg)
        dis2 = dis * dis
        an = dis * abar * jnp.transpose(dis)
        h = x_ref[p]
        for w, bb in ((w1, b1), (w2, b2), (w3, b3)):
            z = jnp.dot(h, w, preferred_element_type=jnp.float32)
            h = jnp.maximum(
                jnp.dot(an, z, preferred_element_type=jnp.float32)
                + dis2 * z + bb, 0.0)
        g = jnp.max(h, axis=0, keepdims=True)
        out_ref[p:p + 1, :] = (
            jnp.dot(g, plw_ref[...], preferred_element_type=jnp.float32)
            + plb_ref[...])


def _group_body(emb_ref, demo_ref, ge_ref, w1a_ref, w1b_ref, b1_ref,
                w2_ref, b2_ref, w3_ref, b3_ref, w4_ref, b4_ref, out_ref):
    s_row = ge_ref[0:1, :]
    d_row = ge_ref[1:2, :]
    iota_b = lax.broadcasted_iota(jnp.int32, (B, GE), 0)
    od = (d_row == iota_b).astype(jnp.float32)
    os_ = (s_row == iota_b).astype(jnp.float32)
    abar = lax.dot_general(od, os_, (((1,), (1,)), ((), ())),
                           preferred_element_type=jnp.float32)
    deg = jnp.sum(abar, axis=1, keepdims=True) + 1.0
    dis = lax.rsqrt(deg)
    dis2 = dis * dis
    an = dis * abar * jnp.transpose(dis)

    z = (jnp.dot(emb_ref[...], w1a_ref[...], preferred_element_type=jnp.float32)
         + jnp.dot(demo_ref[...], w1b_ref[...], preferred_element_type=jnp.float32))
    h = jnp.maximum(jnp.dot(an, z, preferred_element_type=jnp.float32)
                    + dis2 * z + b1_ref[...], 0.0)
    for w_ref, b_ref, act in ((w2_ref, b2_ref, True), (w3_ref, b3_ref, True),
                              (w4_ref, b4_ref, False)):
        z = jnp.dot(h, w_ref[...], preferred_element_type=jnp.float32)
        h = (jnp.dot(an, z, preferred_element_type=jnp.float32)
             + dis2 * z + b_ref[...])
        if act:
            h = jnp.maximum(h, 0.0)
    m = jnp.max(h, axis=1, keepdims=True)
    y = h - m
    out_ref[...] = y - jnp.log(jnp.sum(jnp.exp(y), axis=1, keepdims=True))


def kernel(x, demographic, patient_edge_idx, group_edge_idx,
           pW1, pb1, pW2, pb2, pW3, pb3, plinW, plinb,
           gW1, gb1, gW2, gb2, gW3, gb3, gW4, gb4):
    row = lambda v: v.reshape(1, -1)
    fullg = lambda a: pl.BlockSpec(a.shape, lambda i: (0,) * a.ndim)
    full = lambda a: pl.BlockSpec(a.shape, lambda: (0,) * a.ndim)

    wspecs = [fullg(a) for a in (pW1, row(pb1), pW2, row(pb2), pW3, row(pb3),
                                 plinW, row(plinb))]
    embed = pl.pallas_call(
        _patient_body,
        grid=(B // BP,),
        in_specs=[
            pl.BlockSpec((BP, N, F), lambda i: (i, 0, 0)),
            pl.BlockSpec((BP, 2, PE), lambda i: (i, 0, 0)),
        ] + wspecs,
        out_specs=pl.BlockSpec((BP, GED), lambda i: (i, 0)),
        out_shape=jax.ShapeDtypeStruct((B, GED), jnp.float32),
    )(x, patient_edge_idx, pW1, row(pb1), pW2, row(pb2), pW3, row(pb3),
      plinW, row(plinb))

    gw1a = gW1[:GED]
    gw1b = gW1[GED:]
    gargs = (embed, demographic, group_edge_idx, gw1a, gw1b, row(gb1),
             gW2, row(gb2), gW3, row(gb3), gW4, row(gb4))
    out = pl.pallas_call(
        _group_body,
        in_specs=[full(a) for a in gargs],
        out_specs=pl.BlockSpec((B, NCLS), lambda: (0, 0)),
        out_shape=jax.ShapeDtypeStruct((B, NCLS), jnp.float32),
    )(*gargs)
    return out
